# Initial kernel scaffold; baseline (speedup 1.0000x reference)
import jax
import jax.numpy as jnp
from jax.experimental import pallas as pl


def kernel(inputs, weight):
    raise NotImplementedError("write your pallas kernel here")


# speedup vs baseline: 1.5305x; 1.5305x over previous
import functools

import jax
import jax.numpy as jnp
from jax import lax
from jax.experimental import pallas as pl
from jax.experimental.pallas import tpu as pltpu
from jax.experimental.pallas import tpu_sc as plsc

_NE = 8192
_D = 64
_M = 16384
_BM = 512


def _argmin_body(xsq_ref, x_ref, wt_ref, wsq_ref, idx_ref):
    mm = jnp.dot(x_ref[...], wt_ref[...], preferred_element_type=jnp.float32)
    d = (xsq_ref[...] + wsq_ref[...]) - 2.0 * mm
    idx_ref[...] = jnp.argmin(d, axis=1).astype(jnp.int32)


def _argmin_call(xsq, x, wt, wsq):
    return pl.pallas_call(
        _argmin_body,
        grid=(_M // _BM,),
        in_specs=[
            pl.BlockSpec((_BM, 1), lambda i: (i, 0)),
            pl.BlockSpec((_BM, _D), lambda i: (i, 0)),
            pl.BlockSpec((_D, _NE), lambda i: (0, 0)),
            pl.BlockSpec((1, _NE), lambda i: (0, 0)),
        ],
        out_specs=pl.BlockSpec((_BM,), lambda i: (i,)),
        out_shape=jax.ShapeDtypeStruct((_M,), jnp.int32),
    )(xsq, x, wt, wsq)


_DP = 128


def _gather_call(table_pad, idx):
    mesh = plsc.VectorSubcoreMesh(core_axis_name="c", subcore_axis_name="s")
    nw = mesh.num_cores * mesh.num_subcores
    bw = _M // nw
    nch = bw // _DP
    idx3 = idx.reshape(nw, nch, _DP)

    @functools.partial(
        pl.kernel,
        out_type=jax.ShapeDtypeStruct((_M, _DP), jnp.float32),
        mesh=mesh,
        scratch_types=[
            pltpu.VMEM((nch, _DP), jnp.int32),
            pltpu.VMEM((bw, _DP), jnp.float32),
            pltpu.SemaphoreType.DMA,
        ],
    )
    def gk(table_hbm, idx_hbm, out_hbm, idx_v, rows_v, sem):
        wid = lax.axis_index("s") * mesh.num_cores + lax.axis_index("c")
        pltpu.sync_copy(idx_hbm.at[wid], idx_v)
        cps = [
            pltpu.async_copy(
                table_hbm.at[idx_v.at[j]],
                rows_v.at[pl.ds(j * _DP, _DP)],
                sem,
            )
            for j in range(nch)
        ]
        for c in cps:
            c.wait()
        pltpu.sync_copy(rows_v, out_hbm.at[pl.ds(wid * bw, bw)])

    return gk(table_pad, idx3)


def kernel(inputs, weight):
    x = jnp.transpose(inputs, (0, 2, 3, 1)).reshape(_M, _D)
    xsq = jnp.sum(x ** 2, axis=1, keepdims=True)
    wsq = jnp.sum(weight ** 2, axis=1)[None, :]
    idx = _argmin_call(xsq, x, weight.T, wsq)
    table_pad = jnp.pad(weight, ((0, 0), (0, _DP - _D)))
    q = _gather_call(table_pad, idx)[:, :_D]
    return q.reshape(16, 32, 32, _D).transpose(0, 3, 1, 2)


# speedup vs baseline: 1.6951x; 1.1076x over previous
import functools

import jax
import jax.numpy as jnp
from jax import lax
from jax.experimental import pallas as pl
from jax.experimental.pallas import tpu as pltpu
from jax.experimental.pallas import tpu_sc as plsc

_NE = 8192
_D = 64
_M = 16384
_BM = 512


def _argmin_body(xsq_ref, x_ref, wt_ref, wsq_ref, idx_ref):
    x2 = x_ref[...] * -2.0
    mm2 = jnp.dot(x2, wt_ref[...], preferred_element_type=jnp.float32)
    d = (xsq_ref[...] + wsq_ref[...]) + mm2
    idx_ref[...] = jnp.argmin(d, axis=1).astype(jnp.int32)


def _argmin_call(xsq, x, wt, wsq):
    return pl.pallas_call(
        _argmin_body,
        grid=(_M // _BM,),
        in_specs=[
            pl.BlockSpec((_BM, 1), lambda i: (i, 0)),
            pl.BlockSpec((_BM, _D), lambda i: (i, 0)),
            pl.BlockSpec((_D, _NE), lambda i: (0, 0)),
            pl.BlockSpec((1, _NE), lambda i: (0, 0)),
        ],
        out_specs=pl.BlockSpec((_BM,), lambda i: (i,)),
        out_shape=jax.ShapeDtypeStruct((_M,), jnp.int32),
    )(xsq, x, wt, wsq)


_DP = 128


def _gather_call(table_pad, idx):
    mesh = plsc.VectorSubcoreMesh(core_axis_name="c", subcore_axis_name="s")
    nw = mesh.num_cores * mesh.num_subcores
    bw = _M // nw
    nch = bw // _DP
    idx3 = idx.reshape(nw, nch, _DP)

    @functools.partial(
        pl.kernel,
        out_type=jax.ShapeDtypeStruct((_M, _DP), jnp.float32),
        mesh=mesh,
        scratch_types=[
            pltpu.VMEM((nch, _DP), jnp.int32),
            pltpu.VMEM((bw, _DP), jnp.float32),
            pltpu.SemaphoreType.DMA,
        ],
    )
    def gk(table_hbm, idx_hbm, out_hbm, idx_v, rows_v, sem):
        wid = lax.axis_index("s") * mesh.num_cores + lax.axis_index("c")
        pltpu.sync_copy(idx_hbm.at[wid], idx_v)
        cps = [
            pltpu.async_copy(
                table_hbm.at[idx_v.at[j]],
                rows_v.at[pl.ds(j * _DP, _DP)],
                sem,
            )
            for j in range(nch)
        ]
        for c in cps:
            c.wait()
        pltpu.sync_copy(rows_v, out_hbm.at[pl.ds(wid * bw, bw)])

    return gk(table_pad, idx3)


def kernel(inputs, weight):
    x = jnp.transpose(inputs, (0, 2, 3, 1)).reshape(_M, _D)
    xsq = jnp.sum(x ** 2, axis=1, keepdims=True)
    wsq = jnp.sum(weight ** 2, axis=1)[None, :]
    idx = _argmin_call(xsq, x, weight.T, wsq)
    table_pad = jnp.pad(weight, ((0, 0), (0, _DP - _D)))
    q = _gather_call(table_pad, idx)[:, :_D]
    return q.reshape(16, 32, 32, _D).transpose(0, 3, 1, 2)


# speedup vs baseline: 1.8810x; 1.1097x over previous
import functools

import jax
import jax.numpy as jnp
from jax import lax
from jax.experimental import pallas as pl
from jax.experimental.pallas import tpu as pltpu
from jax.experimental.pallas import tpu_sc as plsc

_NE = 8192
_D = 64
_M = 16384
_BM = 512


def _argmin_body(x_ref, w_ref, wsq_ref, idx_ref):
    x = x_ref[0]
    x2 = x * -2.0
    xsq = jnp.sum(x * x, axis=0, keepdims=True)
    mm2 = jax.lax.dot_general(
        w_ref[...], x2, (((1,), (0,)), ((), ())),
        preferred_element_type=jnp.float32,
    )
    d = (wsq_ref[...] + xsq) + mm2
    idx_ref[...] = jnp.argmin(d, axis=0).astype(jnp.int32)


def _argmin_call(x3, w, wsq):
    nimg, _, npix = x3.shape
    per = npix // _BM
    return pl.pallas_call(
        _argmin_body,
        grid=(_M // _BM,),
        in_specs=[
            pl.BlockSpec((1, _D, _BM), lambda i: (i // per, 0, i % per)),
            pl.BlockSpec((_NE, _D), lambda i: (0, 0)),
            pl.BlockSpec((_NE, 1), lambda i: (0, 0)),
        ],
        out_specs=pl.BlockSpec((_BM,), lambda i: (i,)),
        out_shape=jax.ShapeDtypeStruct((_M,), jnp.int32),
    )(x3, w, wsq)


_DP = 128


def _gather_call(table_pad, idx):
    mesh = plsc.VectorSubcoreMesh(core_axis_name="c", subcore_axis_name="s")
    nw = mesh.num_cores * mesh.num_subcores
    bw = _M // nw
    nch = bw // _DP
    idx3 = idx.reshape(nw, nch, _DP)

    @functools.partial(
        pl.kernel,
        out_type=jax.ShapeDtypeStruct((_M, _DP), jnp.float32),
        mesh=mesh,
        scratch_types=[
            pltpu.VMEM((nch, _DP), jnp.int32),
            pltpu.VMEM((bw, _DP), jnp.float32),
            pltpu.SemaphoreType.DMA,
        ],
    )
    def gk(table_hbm, idx_hbm, out_hbm, idx_v, rows_v, sem):
        wid = lax.axis_index("s") * mesh.num_cores + lax.axis_index("c")
        pltpu.sync_copy(idx_hbm.at[wid], idx_v)
        cps = [
            pltpu.async_copy(
                table_hbm.at[idx_v.at[j]],
                rows_v.at[pl.ds(j * _DP, _DP)],
                sem,
            )
            for j in range(nch)
        ]
        for c in cps:
            c.wait()
        pltpu.sync_copy(rows_v, out_hbm.at[pl.ds(wid * bw, bw)])

    return gk(table_pad, idx3)


def kernel(inputs, weight):
    x3 = inputs.reshape(16, _D, 32 * 32)
    wsq = jnp.sum(weight ** 2, axis=1)[:, None]
    idx = _argmin_call(x3, weight, wsq)
    table_pad = jnp.pad(weight, ((0, 0), (0, _DP - _D)))
    q = _gather_call(table_pad, idx)[:, :_D]
    return q.reshape(16, 32, 32, _D).transpose(0, 3, 1, 2)


# speedup vs baseline: 1.8924x; 1.0061x over previous
import functools

import jax
import jax.numpy as jnp
from jax import lax
from jax.experimental import pallas as pl
from jax.experimental.pallas import tpu as pltpu
from jax.experimental.pallas import tpu_sc as plsc

_NE = 8192
_D = 64
_M = 16384
_BM = 512
_DP = 128


def _argmin_body(x_ref, w_ref, wsq_ref, idx_ref, wpad_ref):
    x = x_ref[0]
    x2 = x * -2.0
    xsq = jnp.sum(x * x, axis=0, keepdims=True)
    mm2 = jax.lax.dot_general(
        w_ref[...], x2, (((1,), (0,)), ((), ())),
        preferred_element_type=jnp.float32,
    )
    d = (wsq_ref[...] + xsq) + mm2
    idx_ref[...] = jnp.argmin(d, axis=0).astype(jnp.int32)

    @pl.when(pl.program_id(0) == 0)
    def _():
        wpad_ref[:, :_D] = w_ref[...]
        wpad_ref[:, _D:] = jnp.zeros((_NE, _DP - _D), jnp.float32)


def _argmin_call(x3, w, wsq):
    nimg, _, npix = x3.shape
    per = npix // _BM
    return pl.pallas_call(
        _argmin_body,
        grid=(_M // _BM,),
        in_specs=[
            pl.BlockSpec((1, _D, _BM), lambda i: (i // per, 0, i % per)),
            pl.BlockSpec((_NE, _D), lambda i: (0, 0)),
            pl.BlockSpec((_NE, 1), lambda i: (0, 0)),
        ],
        out_specs=[
            pl.BlockSpec((_BM,), lambda i: (i,)),
            pl.BlockSpec((_NE, _DP), lambda i: (0, 0)),
        ],
        out_shape=[
            jax.ShapeDtypeStruct((_M,), jnp.int32),
            jax.ShapeDtypeStruct((_NE, _DP), jnp.float32),
        ],
    )(x3, w, wsq)


def _gather_call(table_pad, idx):
    mesh = plsc.VectorSubcoreMesh(core_axis_name="c", subcore_axis_name="s")
    nw = mesh.num_cores * mesh.num_subcores
    bw = _M // nw
    nch = bw // _DP
    idx3 = idx.reshape(nw, nch, _DP)

    @functools.partial(
        pl.kernel,
        out_type=jax.ShapeDtypeStruct((_M, _DP), jnp.float32),
        mesh=mesh,
        scratch_types=[
            pltpu.VMEM((nch, _DP), jnp.int32),
            pltpu.VMEM((bw, _DP), jnp.float32),
            pltpu.SemaphoreType.DMA,
        ],
    )
    def gk(table_hbm, idx_hbm, out_hbm, idx_v, rows_v, sem):
        wid = lax.axis_index("s") * mesh.num_cores + lax.axis_index("c")
        pltpu.sync_copy(idx_hbm.at[wid], idx_v)
        cps = [
            pltpu.async_copy(
                table_hbm.at[idx_v.at[j]],
                rows_v.at[pl.ds(j * _DP, _DP)],
                sem,
            )
            for j in range(nch)
        ]
        for c in cps:
            c.wait()
        pltpu.sync_copy(rows_v, out_hbm.at[pl.ds(wid * bw, bw)])

    return gk(table_pad, idx3)


def kernel(inputs, weight):
    x3 = inputs.reshape(16, _D, 32 * 32)
    wsq = jnp.sum(weight ** 2, axis=1)[:, None]
    idx, table_pad = _argmin_call(x3, weight, wsq)
    q = _gather_call(table_pad, idx)[:, :_D]
    return q.reshape(16, 32, 32, _D).transpose(0, 3, 1, 2)


# speedup vs baseline: 1.9969x; 1.0552x over previous
import functools

import jax
import jax.numpy as jnp
from jax import lax
from jax.experimental import pallas as pl
from jax.experimental.pallas import tpu as pltpu
from jax.experimental.pallas import tpu_sc as plsc

_NE = 8192
_D = 64
_M = 16384
_BM = 1024
_DP = 128


def _argmin_body(x_ref, w_ref, wsq_ref, idx_ref, wpad_ref):
    x = x_ref[0]
    x2 = x * -2.0
    xsq = jnp.sum(x * x, axis=0, keepdims=True)
    mm2 = jax.lax.dot_general(
        w_ref[...], x2, (((1,), (0,)), ((), ())),
        preferred_element_type=jnp.float32,
    )
    d = (wsq_ref[...] + xsq) + mm2
    idx_ref[...] = jnp.argmin(d, axis=0).astype(jnp.int32)

    @pl.when(pl.program_id(0) == 0)
    def _():
        wpad_ref[:, :_D] = w_ref[...]
        wpad_ref[:, _D:] = jnp.zeros((_NE, _DP - _D), jnp.float32)


def _argmin_call(x3, w, wsq):
    nimg, _, npix = x3.shape
    per = npix // _BM
    return pl.pallas_call(
        _argmin_body,
        grid=(_M // _BM,),
        in_specs=[
            pl.BlockSpec((1, _D, _BM), lambda i: (i // per, 0, i % per)),
            pl.BlockSpec((_NE, _D), lambda i: (0, 0)),
            pl.BlockSpec((_NE, 1), lambda i: (0, 0)),
        ],
        out_specs=[
            pl.BlockSpec((_BM,), lambda i: (i,)),
            pl.BlockSpec((_NE, _DP), lambda i: (0, 0)),
        ],
        out_shape=[
            jax.ShapeDtypeStruct((_M,), jnp.int32),
            jax.ShapeDtypeStruct((_NE, _DP), jnp.float32),
        ],
    )(x3, w, wsq)


def _gather_call(table_pad, idx):
    mesh = plsc.VectorSubcoreMesh(core_axis_name="c", subcore_axis_name="s")
    nw = mesh.num_cores * mesh.num_subcores
    bw = _M // nw
    nch = bw // _DP
    idx3 = idx.reshape(nw, nch, _DP)

    @functools.partial(
        pl.kernel,
        out_type=jax.ShapeDtypeStruct((_M, _DP), jnp.float32),
        mesh=mesh,
        scratch_types=[
            pltpu.VMEM((nch, _DP), jnp.int32),
            pltpu.VMEM((bw, _DP), jnp.float32),
            pltpu.SemaphoreType.DMA,
        ],
    )
    def gk(table_hbm, idx_hbm, out_hbm, idx_v, rows_v, sem):
        wid = lax.axis_index("s") * mesh.num_cores + lax.axis_index("c")
        pltpu.sync_copy(idx_hbm.at[wid], idx_v)
        cps = [
            pltpu.async_copy(
                table_hbm.at[idx_v.at[j]],
                rows_v.at[pl.ds(j * _DP, _DP)],
                sem,
            )
            for j in range(nch)
        ]
        for c in cps:
            c.wait()
        pltpu.sync_copy(rows_v, out_hbm.at[pl.ds(wid * bw, bw)])

    return gk(table_pad, idx3)


def kernel(inputs, weight):
    x3 = inputs.reshape(16, _D, 32 * 32)
    wsq = jnp.sum(weight ** 2, axis=1)[:, None]
    idx, table_pad = _argmin_call(x3, weight, wsq)
    q = _gather_call(table_pad, idx)[:, :_D]
    return q.reshape(16, 32, 32, _D).transpose(0, 3, 1, 2)
